# Initial kernel scaffold; baseline (speedup 1.0000x reference)
import functools

import jax
import jax.numpy as jnp
from jax import lax
from jax.experimental import pallas as pl
from jax.experimental.pallas import tpu as pltpu
from jax.experimental.pallas import tpu_sc as plsc

N = 10000
E = 320000
D = 128
XW = 16

NC = 2
NS = 16
NW = NC * NS
EPT = E // NW
CH = 80
NCHUNK = EPT // CH
NPT = N // NS

BN = 2000
BE = 2000

_mesh = plsc.VectorSubcoreMesh(core_axis_name="c", subcore_axis_name="s",
                               num_cores=NC, num_subcores=NS)

f32 = jnp.float32


def _silu(x):
    return x * jax.nn.sigmoid(x)


@functools.partial(
    pl.kernel,
    out_type=[
        jax.ShapeDtypeStruct((E, D), f32),
        jax.ShapeDtypeStruct((E, D), f32),
        jax.ShapeDtypeStruct((E, XW), f32),
        jax.ShapeDtypeStruct((E, XW), f32),
    ],
    mesh=_mesh,
    scratch_types=[
        pltpu.VMEM((CH,), jnp.int32),
        pltpu.VMEM((CH,), jnp.int32),
        pltpu.VMEM((CH, D), f32),
        pltpu.VMEM((CH, D), f32),
        pltpu.VMEM((CH, XW), f32),
        pltpu.VMEM((CH, XW), f32),
        pltpu.SemaphoreType.DMA,
        pltpu.SemaphoreType.DMA,
        pltpu.SemaphoreType.DMA,
        pltpu.SemaphoreType.DMA,
    ],
)
def _sc_gather(p_hbm, q_hbm, x_hbm, src_hbm, dst_hbm,
               pg_out, qg_out, xs_out, xd_out,
               idxs_v, idxd_v, bufp, bufq, bufxs, bufxd,
               sem0, sem1, sem2, sem3):
    wid = lax.axis_index("s") * NC + lax.axis_index("c")
    base = wid * EPT

    def body(k, carry):
        off = pl.multiple_of(base + k * CH, 8)
        pltpu.sync_copy(src_hbm.at[pl.ds(off, CH)], idxs_v)
        pltpu.sync_copy(dst_hbm.at[pl.ds(off, CH)], idxd_v)
        c0 = pltpu.async_copy(p_hbm.at[idxs_v], bufp, sem0)
        c1 = pltpu.async_copy(q_hbm.at[idxd_v], bufq, sem1)
        c2 = pltpu.async_copy(x_hbm.at[idxs_v], bufxs, sem2)
        c3 = pltpu.async_copy(x_hbm.at[idxd_v], bufxd, sem3)
        c0.wait()
        c1.wait()
        c2.wait()
        c3.wait()
        pltpu.sync_copy(bufp, pg_out.at[pl.ds(off, CH)])
        pltpu.sync_copy(bufq, qg_out.at[pl.ds(off, CH)])
        pltpu.sync_copy(bufxs, xs_out.at[pl.ds(off, CH)])
        pltpu.sync_copy(bufxd, xd_out.at[pl.ds(off, CH)])
        return carry

    lax.fori_loop(0, NCHUNK, body, 0)


def _make_sc_scatter(with_m: bool):
    out_type = []
    if with_m:
        out_type.append(jax.ShapeDtypeStruct((NC, N, D), f32))
    out_type.append(jax.ShapeDtypeStruct((NC, N, XW), f32))
    scratch = [pltpu.VMEM((CH,), jnp.int32)]
    if with_m:
        scratch.append(pltpu.VMEM((CH, D), f32))
        scratch.append(pltpu.VMEM_SHARED((N, D), f32))
    scratch.append(pltpu.VMEM((CH, XW), f32))
    scratch.append(pltpu.VMEM_SHARED((N, XW), f32))

    def body_full(m_hbm, c_hbm, dst_hbm, z128_hbm, z16_hbm,
                  hagg_out, xagg_out, idx_v, bufm, hagg_s, bufc, xagg_s):
        cid = lax.axis_index("c")
        sid = lax.axis_index("s")
        rows = pl.ds(sid * NPT, NPT)
        pltpu.sync_copy(z128_hbm.at[rows], hagg_s.at[rows])
        pltpu.sync_copy(z16_hbm.at[rows], xagg_s.at[rows])
        plsc.subcore_barrier()
        base = cid * (E // NC) + sid * EPT

        def chunk(k, carry):
            off = pl.multiple_of(base + k * CH, 8)
            pltpu.sync_copy(dst_hbm.at[pl.ds(off, CH)], idx_v)
            pltpu.sync_copy(m_hbm.at[pl.ds(off, CH)], bufm)
            pltpu.sync_copy(c_hbm.at[pl.ds(off, CH)], bufc)
            pltpu.sync_copy(bufm, hagg_s.at[idx_v], add=True)
            pltpu.sync_copy(bufc, xagg_s.at[idx_v], add=True)
            return carry

        lax.fori_loop(0, NCHUNK, chunk, 0)
        plsc.subcore_barrier()
        pltpu.sync_copy(hagg_s.at[rows], hagg_out.at[cid, rows])
        pltpu.sync_copy(xagg_s.at[rows], xagg_out.at[cid, rows])

    def body_x(c_hbm, dst_hbm, z16_hbm, xagg_out, idx_v, bufc, xagg_s):
        cid = lax.axis_index("c")
        sid = lax.axis_index("s")
        rows = pl.ds(sid * NPT, NPT)
        pltpu.sync_copy(z16_hbm.at[rows], xagg_s.at[rows])
        plsc.subcore_barrier()
        base = cid * (E // NC) + sid * EPT

        def chunk(k, carry):
            off = pl.multiple_of(base + k * CH, 8)
            pltpu.sync_copy(dst_hbm.at[pl.ds(off, CH)], idx_v)
            pltpu.sync_copy(c_hbm.at[pl.ds(off, CH)], bufc)
            pltpu.sync_copy(bufc, xagg_s.at[idx_v], add=True)
            return carry

        lax.fori_loop(0, NCHUNK, chunk, 0)
        plsc.subcore_barrier()
        pltpu.sync_copy(xagg_s.at[rows], xagg_out.at[cid, rows])

    return pl.kernel(body_full if with_m else body_x, out_type=out_type,
                     mesh=_mesh, scratch_types=scratch)


_sc_scatter_full = _make_sc_scatter(True)
_sc_scatter_x = _make_sc_scatter(False)


def _node0_body(ids_ref, rest_ref, wp_ref, bp_ref, wpq_ref, pqb_ref,
                h_ref, p_ref, q_ref):
    ids = ids_ref[...]
    lane = lax.broadcasted_iota(jnp.int32, (BN, 16), 1)
    oh = (ids == lane).astype(f32)
    feat = jnp.concatenate([oh, rest_ref[...]], axis=1)
    h = jnp.dot(feat, wp_ref[...], preferred_element_type=f32) + bp_ref[...]
    pq = jnp.dot(h, wpq_ref[...], preferred_element_type=f32) + pqb_ref[...]
    h_ref[...] = h
    p_ref[...] = pq[:, :D]
    q_ref[...] = pq[:, D:]


def _nodei_body(h_ref, hagg_ref, x_ref, xagg_ref,
                wh1a_ref, wh1b_ref, bh1_ref, wh2_ref, bh2_ref, wpq_ref, pqb_ref,
                h_out, x_out, p_ref, q_ref):
    h = h_ref[...]
    agg = hagg_ref[0] + hagg_ref[1]
    u = _silu(jnp.dot(h, wh1a_ref[...], preferred_element_type=f32)
              + jnp.dot(agg, wh1b_ref[...], preferred_element_type=f32)
              + bh1_ref[...])
    hn = h + jnp.dot(u, wh2_ref[...], preferred_element_type=f32) + bh2_ref[...]
    xn = x_ref[...] + (xagg_ref[0] + xagg_ref[1]) * (1.0 / 32.0)
    pq = jnp.dot(hn, wpq_ref[...], preferred_element_type=f32) + pqb_ref[...]
    h_out[...] = hn
    x_out[...] = xn
    p_ref[...] = pq[:, :D]
    q_ref[...] = pq[:, D:]


def _edge_body(first, last, *refs):
    if first:
        (pg_ref, qg_ref, xs_ref, xd_ref, w257_ref, we2_ref, be2_ref, wx_ref,
         bx_ref, *outs) = refs
    else:
        (pg_ref, qg_ref, xs_ref, xd_ref, a_ref, w257_ref, we2_ref, be2_ref,
         wx_ref, bx_ref, *outs) = refs
    d = xs_ref[...] - xd_ref[...]
    dn2 = jnp.sum(d * d, axis=1, keepdims=True)
    if first:
        a = jnp.sqrt(dn2)
    else:
        a = a_ref[:, 0:1]
    pre = pg_ref[...] + qg_ref[...] + a * w257_ref[...]
    u = _silu(pre)
    m = _silu(jnp.dot(u, we2_ref[...], preferred_element_type=f32) + be2_ref[...])
    coef = jnp.sum(m * wx_ref[...], axis=1, keepdims=True) + bx_ref[0, 0]
    dn = jnp.sqrt(dn2) + 1.0
    c16 = d * (coef / (dn * 32.0))
    oi = 0
    if not last:
        outs[oi][...] = m
        oi += 1
    outs[oi][...] = c16
    oi += 1
    if first:
        outs[oi][...] = jnp.broadcast_to(a, (BE, XW))


def _final_body(x_ref, xagg_ref, xyz_ref, mass_ref, out_ref):
    x4 = x_ref[...] + (xagg_ref[0] + xagg_ref[1]) * (1.0 / 32.0)
    vel = x4 - xyz_ref[...]
    w = mass_ref[:, 0:1]
    com = jnp.sum(vel * w, axis=0, keepdims=True) / jnp.sum(w)
    out_ref[...] = vel - com


def _full(shape):
    return pl.BlockSpec(shape, lambda i: tuple(0 for _ in shape))


def _rows(shape):
    return pl.BlockSpec(shape, lambda i: (i,) + tuple(0 for _ in shape[1:]))


def _tc_node0(ids16, rest16, wp32, bp, wpq, pqb):
    return pl.pallas_call(
        _node0_body,
        grid=(N // BN,),
        in_specs=[_rows((BN, 16)), _rows((BN, 16)), _full((32, D)),
                  _full((1, D)), _full((D, 2 * D)), _full((1, 2 * D))],
        out_specs=[_rows((BN, D)), _rows((BN, D)), _rows((BN, D))],
        out_shape=[jax.ShapeDtypeStruct((N, D), f32)] * 3,
    )(ids16, rest16, wp32, bp, wpq, pqb)


def _tc_nodei(h, hagg, x, xagg, wh1a, wh1b, bh1, wh2, bh2, wpq, pqb):
    return pl.pallas_call(
        _nodei_body,
        grid=(N // BN,),
        in_specs=[_rows((BN, D)),
                  pl.BlockSpec((NC, BN, D), lambda i: (0, i, 0)),
                  _rows((BN, XW)),
                  pl.BlockSpec((NC, BN, XW), lambda i: (0, i, 0)),
                  _full((D, D)), _full((D, D)), _full((1, D)),
                  _full((D, D)), _full((1, D)),
                  _full((D, 2 * D)), _full((1, 2 * D))],
        out_specs=[_rows((BN, D)), _rows((BN, XW)), _rows((BN, D)),
                   _rows((BN, D))],
        out_shape=[jax.ShapeDtypeStruct((N, D), f32),
                   jax.ShapeDtypeStruct((N, XW), f32),
                   jax.ShapeDtypeStruct((N, D), f32),
                   jax.ShapeDtypeStruct((N, D), f32)],
    )(h, hagg, x, xagg, wh1a, wh1b, bh1, wh2, bh2, wpq, pqb)


def _tc_edge(first, last, pg, qg, xs, xd, a16, w257, we2, be2, wx, bx):
    in_specs = [_rows((BE, D)), _rows((BE, D)), _rows((BE, XW)),
                _rows((BE, XW))]
    args = [pg, qg, xs, xd]
    if not first:
        in_specs.append(_rows((BE, XW)))
        args.append(a16)
    in_specs += [_full((1, D)), _full((D, D)), _full((1, D)), _full((1, D)),
                 _full((1, 1))]
    args += [w257, we2, be2, wx, bx]
    out_specs, out_shape = [], []
    if not last:
        out_specs.append(_rows((BE, D)))
        out_shape.append(jax.ShapeDtypeStruct((E, D), f32))
    out_specs.append(_rows((BE, XW)))
    out_shape.append(jax.ShapeDtypeStruct((E, XW), f32))
    if first:
        out_specs.append(_rows((BE, XW)))
        out_shape.append(jax.ShapeDtypeStruct((E, XW), f32))
    return pl.pallas_call(
        functools.partial(_edge_body, first, last),
        grid=(E // BE,),
        in_specs=in_specs,
        out_specs=out_specs,
        out_shape=out_shape,
    )(*args)


def _tc_final(x3, xagg, xyz16, mass16):
    return pl.pallas_call(
        _final_body,
        grid=(1,),
        in_specs=[_rows((N, XW)),
                  pl.BlockSpec((NC, N, XW), lambda i: (0, 0, 0)),
                  _rows((N, XW)), _rows((N, XW))],
        out_specs=_rows((N, XW)),
        out_shape=jax.ShapeDtypeStruct((N, XW), f32),
    )(x3, xagg, xyz16, mass16)


def kernel(xyz, atom_ids, atom_masses, cond_labels, cond_mask, moments,
           edge_index, t, Wp, bp, We1, be1, We2, be2, Wx, bx, Wh1, bh1, Wh2, bh2):
    L = We1.shape[0]
    src = edge_index[0]
    dst = edge_index[1]

    ids16 = jnp.broadcast_to(atom_ids.astype(jnp.int32), (N, 16))
    temb = jnp.broadcast_to(t.reshape(1, 1), (N, 1))
    rest = jnp.concatenate([atom_masses / 12.0, temb, cond_mask, cond_labels,
                            moments / (float(N) * 12.0)], axis=1)
    rest16 = jnp.pad(rest, ((0, 0), (0, 16 - rest.shape[1])))
    wp32 = jnp.pad(Wp, ((0, 32 - Wp.shape[0]), (0, 0)))
    xyz16 = jnp.pad(xyz, ((0, 0), (0, XW - 3)))
    mass16 = jnp.pad(atom_masses, ((0, 0), (0, XW - 1)))
    z128 = jnp.zeros((N, D), f32)
    z16 = jnp.zeros((N, XW), f32)

    def wpq(i):
        return (jnp.concatenate([We1[i, :D, :], We1[i, D:2 * D, :]], axis=1),
                jnp.concatenate([jnp.zeros((1, D), f32), be1[i].reshape(1, D)],
                                axis=1))

    w0, b0 = wpq(0)
    h, P, Q = _tc_node0(ids16, rest16, wp32, bp.reshape(1, D), w0, b0)
    x = xyz16
    a16 = None
    out16 = None
    for i in range(L):
        first, last = i == 0, i == L - 1
        Pg, Qg, xs, xd = _sc_gather(P, Q, x, src, dst)
        res = _tc_edge(first, last, Pg, Qg, xs, xd, a16,
                       We1[i, 2 * D].reshape(1, D), We2[i],
                       be2[i].reshape(1, D), Wx[i].reshape(1, D),
                       bx[i].reshape(1, 1))
        ri = 0
        if not last:
            m = res[ri]
            ri += 1
        c16 = res[ri]
        ri += 1
        if first:
            a16 = res[ri]
        if not last:
            hagg, xagg = _sc_scatter_full(m, c16, dst, z128, z16)
            h, x, P, Q = _tc_nodei(h, hagg, x, xagg,
                                   Wh1[i, :D], Wh1[i, D:], bh1[i].reshape(1, D),
                                   Wh2[i], bh2[i].reshape(1, D), *wpq(i + 1))
        else:
            xagg = _sc_scatter_x(c16, dst, z16)
            if isinstance(xagg, (list, tuple)):
                xagg = xagg[0]
            out16 = _tc_final(x, xagg, xyz16, mass16)
    return out16[:, :3]


# speedup vs baseline: 2.9811x; 2.9811x over previous
import functools

import jax
import jax.numpy as jnp
from jax import lax
from jax.experimental import pallas as pl
from jax.experimental.pallas import tpu as pltpu
from jax.experimental.pallas import tpu_sc as plsc

N = 10000
E = 320000
D = 128
XW = 16

NC = 2
NS = 16
NW = NC * NS
EPT = E // NW
CH = 80
NG = CH // 16
NCHUNK = EPT // CH
NP = 10240
NPT = NP // NS

BN = 2000
BE = 2000

f32 = jnp.float32
i32 = jnp.int32


@functools.lru_cache(maxsize=None)
def _mesh():
    return plsc.VectorSubcoreMesh(core_axis_name="c", subcore_axis_name="s",
                                  num_cores=NC, num_subcores=NS)


def _silu(x):
    return x * jax.nn.sigmoid(x)


def _sc_gather_body(p_hbm, q_hbm, xcx_hbm, xcy_hbm, xcz_hbm,
                    xyx_hbm, xyy_hbm, xyz_hbm, src_hbm, dst_hbm,
                    pg_out, qg_out, d_out,
                    idxs_v, idxd_v, bufp, bufq, bufd,
                    xc0, xc1, xc2, xy0, xy1, xy2,
                    semp, semq):
    wid = lax.axis_index("s") * NC + lax.axis_index("c")
    base = wid * EPT
    coords = (xc0, xc1, xc2)
    coords0 = (xy0, xy1, xy2)
    for c, src_c in enumerate((xcx_hbm, xcy_hbm, xcz_hbm)):
        pltpu.sync_copy(src_c, coords[c])
    for c, src_c in enumerate((xyx_hbm, xyy_hbm, xyz_hbm)):
        pltpu.sync_copy(src_c, coords0[c])
    lane = lax.iota(i32, 16)

    def body(k, carry):
        off = pl.multiple_of(base + k * CH, 8)
        pltpu.sync_copy(src_hbm.at[pl.ds(off, CH)], idxs_v)
        pltpu.sync_copy(dst_hbm.at[pl.ds(off, CH)], idxd_v)
        cp = pltpu.async_copy(p_hbm.at[idxs_v], bufp, semp)
        cq = pltpu.async_copy(q_hbm.at[idxd_v], bufq, semq)
        for g in range(NG):
            ivs = idxs_v[pl.ds(g * 16, 16)]
            ivd = idxd_v[pl.ds(g * 16, 16)]
            row = lane + g * 16
            for c in range(3):
                cvec = jnp.full((16,), c, i32)
                dx = (plsc.load_gather(coords[c], [ivs])
                      - plsc.load_gather(coords[c], [ivd]))
                plsc.store_scatter(bufd, [row, cvec], dx)
                d0 = (plsc.load_gather(coords0[c], [ivs])
                      - plsc.load_gather(coords0[c], [ivd]))
                plsc.store_scatter(bufd, [row, cvec + 4], d0)
        cp.wait()
        cq.wait()
        pltpu.sync_copy(bufp, pg_out.at[pl.ds(off, CH)])
        pltpu.sync_copy(bufq, qg_out.at[pl.ds(off, CH)])
        pltpu.sync_copy(bufd, d_out.at[pl.ds(off, CH)])
        return carry

    lax.fori_loop(0, NCHUNK, body, 0)


@functools.lru_cache(maxsize=None)
def _get_sc_gather():
    return pl.kernel(
        _sc_gather_body,
        out_type=[
            jax.ShapeDtypeStruct((E, D), f32),
            jax.ShapeDtypeStruct((E, D), f32),
            jax.ShapeDtypeStruct((E, XW), f32),
        ],
        mesh=_mesh(),
        scratch_types=[
            pltpu.VMEM((CH,), i32),
            pltpu.VMEM((CH,), i32),
            pltpu.VMEM((CH, D), f32),
            pltpu.VMEM((CH, D), f32),
            pltpu.VMEM((CH, XW), f32),
            pltpu.VMEM((N,), f32),
            pltpu.VMEM((N,), f32),
            pltpu.VMEM((N,), f32),
            pltpu.VMEM((N,), f32),
            pltpu.VMEM((N,), f32),
            pltpu.VMEM((N,), f32),
            pltpu.SemaphoreType.DMA,
            pltpu.SemaphoreType.DMA,
        ],
        compiler_params=pltpu.CompilerParams(needs_layout_passes=False),
    )


@functools.lru_cache(maxsize=None)
def _make_sc_scatter(with_m: bool):
    out_type = []
    if with_m:
        out_type.append(jax.ShapeDtypeStruct((NC, NP, D), f32))
    out_type.append(jax.ShapeDtypeStruct((NC, NP, D), f32))
    scratch = [
        pltpu.VMEM((CH,), i32),
        pltpu.VMEM((CH, D), f32),
        pltpu.VMEM((CH, XW), f32),
        pltpu.VMEM_SHARED((NP, D), f32),
    ]

    def body(*refs):
        if with_m:
            (m_hbm, c_hbm, dst_hbm, z128_hbm,
             hagg_out, xagg_out, idx_v, bufm, bufc, agg_s) = refs
        else:
            (c_hbm, dst_hbm, z128_hbm,
             xagg_out, idx_v, bufm, bufc, agg_s) = refs
        cid = lax.axis_index("c")
        sid = lax.axis_index("s")
        rows = pl.ds(sid * NPT, NPT)
        base = cid * (E // NC) + sid * EPT

        if with_m:
            pltpu.sync_copy(z128_hbm.at[rows], agg_s.at[rows])
            plsc.subcore_barrier()

            def chunk_m(k, carry):
                off = pl.multiple_of(base + k * CH, 8)
                pltpu.sync_copy(dst_hbm.at[pl.ds(off, CH)], idx_v)
                pltpu.sync_copy(m_hbm.at[pl.ds(off, CH)], bufm)
                pltpu.sync_copy(bufm, agg_s.at[idx_v], add=True)
                return carry

            lax.fori_loop(0, NCHUNK, chunk_m, 0)
            plsc.subcore_barrier()
            pltpu.sync_copy(agg_s.at[rows], hagg_out.at[cid, rows])

        pltpu.sync_copy(z128_hbm.at[rows], agg_s.at[rows])
        pltpu.sync_copy(z128_hbm.at[pl.ds(0, CH)], bufm)
        plsc.subcore_barrier()

        def chunk_x(k, carry):
            off = pl.multiple_of(base + k * CH, 8)
            pltpu.sync_copy(dst_hbm.at[pl.ds(off, CH)], idx_v)
            pltpu.sync_copy(c_hbm.at[pl.ds(off, CH)], bufc)
            for e in range(CH):
                bufm[e, pl.ds(0, XW)] = bufc[e, :]
            pltpu.sync_copy(bufm, agg_s.at[idx_v], add=True)
            return carry

        lax.fori_loop(0, NCHUNK, chunk_x, 0)
        plsc.subcore_barrier()
        pltpu.sync_copy(agg_s.at[rows], xagg_out.at[cid, rows])

    return pl.kernel(body, out_type=out_type, mesh=_mesh(),
                     scratch_types=scratch)


def _sc_gather(*args):
    return _get_sc_gather()(*args)


def _sc_scatter_full(*args):
    return _make_sc_scatter(True)(*args)


def _sc_scatter_x(*args):
    res = _make_sc_scatter(False)(*args)
    return res[0] if isinstance(res, (list, tuple)) else res


def _node0_body(ids_ref, rest_ref, wp_ref, bp_ref, wpq_ref, pqb_ref,
                h_ref, p_ref, q_ref):
    ids = ids_ref[...]
    lane = lax.broadcasted_iota(i32, (BN, 16), 1)
    oh = (ids == lane).astype(f32)
    feat = jnp.concatenate([oh, rest_ref[...]], axis=1)
    h = jnp.dot(feat, wp_ref[...], preferred_element_type=f32) + bp_ref[...]
    pq = jnp.dot(h, wpq_ref[...], preferred_element_type=f32) + pqb_ref[...]
    h_ref[...] = h
    p_ref[...] = pq[:, :D]
    q_ref[...] = pq[:, D:]


def _nodei_body(h_ref, hagg_ref, x_ref, xagg_ref,
                wh1a_ref, wh1b_ref, bh1_ref, wh2_ref, bh2_ref, wpq_ref, pqb_ref,
                h_out, x_out, p_ref, q_ref):
    h = h_ref[...]
    agg = hagg_ref[0] + hagg_ref[1]
    u = _silu(jnp.dot(h, wh1a_ref[...], preferred_element_type=f32)
              + jnp.dot(agg, wh1b_ref[...], preferred_element_type=f32)
              + bh1_ref[...])
    hn = h + jnp.dot(u, wh2_ref[...], preferred_element_type=f32) + bh2_ref[...]
    xn = x_ref[...] + (xagg_ref[0] + xagg_ref[1])[:, :XW]
    pq = jnp.dot(hn, wpq_ref[...], preferred_element_type=f32) + pqb_ref[...]
    h_out[...] = hn
    x_out[...] = xn
    p_ref[...] = pq[:, :D]
    q_ref[...] = pq[:, D:]


def _edge_body(last, pg_ref, qg_ref, d_ref, w257_ref, we2_ref, be2_ref,
               wx_ref, bx_ref, *outs):
    draw = d_ref[...]
    lane = lax.broadcasted_iota(i32, (BE, XW), 1)
    d = jnp.where(lane < 3, draw, 0.0)
    d0 = jnp.where((lane >= 4) & (lane < 7), draw, 0.0)
    dn2 = jnp.sum(d * d, axis=1, keepdims=True)
    a = jnp.sqrt(jnp.sum(d0 * d0, axis=1, keepdims=True))
    pre = pg_ref[...] + qg_ref[...] + a * w257_ref[...]
    u = _silu(pre)
    m = _silu(jnp.dot(u, we2_ref[...], preferred_element_type=f32) + be2_ref[...])
    coef = jnp.sum(m * wx_ref[...], axis=1, keepdims=True) + bx_ref[0, 0]
    dn = jnp.sqrt(dn2) + 1.0
    c16 = d * (coef / (dn * 32.0))
    oi = 0
    if not last:
        outs[oi][...] = m
        oi += 1
    outs[oi][...] = c16


def _final_body(x_ref, xagg_ref, xyz_ref, mass_ref, out_ref):
    x4 = x_ref[...] + (xagg_ref[0] + xagg_ref[1])[:, :XW]
    vel = x4 - xyz_ref[...]
    w = mass_ref[:, 0:1]
    com = jnp.sum(vel * w, axis=0, keepdims=True) / jnp.sum(w)
    out_ref[...] = vel - com


def _full(shape):
    return pl.BlockSpec(shape, lambda i: tuple(0 for _ in shape))


def _rows(shape):
    return pl.BlockSpec(shape, lambda i: (i,) + tuple(0 for _ in shape[1:]))


def _tc_node0(ids16, rest16, wp32, bp, wpq, pqb):
    return pl.pallas_call(
        _node0_body,
        grid=(N // BN,),
        in_specs=[_rows((BN, 16)), _rows((BN, 16)), _full((32, D)),
                  _full((1, D)), _full((D, 2 * D)), _full((1, 2 * D))],
        out_specs=[_rows((BN, D)), _rows((BN, D)), _rows((BN, D))],
        out_shape=[jax.ShapeDtypeStruct((N, D), f32)] * 3,
    )(ids16, rest16, wp32, bp, wpq, pqb)


def _tc_nodei(h, hagg, x, xagg, wh1a, wh1b, bh1, wh2, bh2, wpq, pqb):
    return pl.pallas_call(
        _nodei_body,
        grid=(N // BN,),
        in_specs=[_rows((BN, D)),
                  pl.BlockSpec((NC, BN, D), lambda i: (0, i, 0)),
                  _rows((BN, XW)),
                  pl.BlockSpec((NC, BN, D), lambda i: (0, i, 0)),
                  _full((D, D)), _full((D, D)), _full((1, D)),
                  _full((D, D)), _full((1, D)),
                  _full((D, 2 * D)), _full((1, 2 * D))],
        out_specs=[_rows((BN, D)), _rows((BN, XW)), _rows((BN, D)),
                   _rows((BN, D))],
        out_shape=[jax.ShapeDtypeStruct((N, D), f32),
                   jax.ShapeDtypeStruct((N, XW), f32),
                   jax.ShapeDtypeStruct((N, D), f32),
                   jax.ShapeDtypeStruct((N, D), f32)],
    )(h, hagg, x, xagg, wh1a, wh1b, bh1, wh2, bh2, wpq, pqb)


def _tc_edge(last, pg, qg, d16, w257, we2, be2, wx, bx):
    in_specs = [_rows((BE, D)), _rows((BE, D)), _rows((BE, XW)),
                _full((1, D)), _full((D, D)), _full((1, D)), _full((1, D)),
                _full((1, 1))]
    out_specs, out_shape = [], []
    if not last:
        out_specs.append(_rows((BE, D)))
        out_shape.append(jax.ShapeDtypeStruct((E, D), f32))
    out_specs.append(_rows((BE, XW)))
    out_shape.append(jax.ShapeDtypeStruct((E, XW), f32))
    return pl.pallas_call(
        functools.partial(_edge_body, last),
        grid=(E // BE,),
        in_specs=in_specs,
        out_specs=out_specs,
        out_shape=out_shape,
    )(pg, qg, d16, w257, we2, be2, wx, bx)


def _tc_final(x3, xagg, xyz16, mass16):
    return pl.pallas_call(
        _final_body,
        grid=(1,),
        in_specs=[_rows((N, XW)),
                  pl.BlockSpec((NC, N, D), lambda i: (0, 0, 0)),
                  _rows((N, XW)), _rows((N, XW))],
        out_specs=_rows((N, XW)),
        out_shape=jax.ShapeDtypeStruct((N, XW), f32),
    )(x3, xagg, xyz16, mass16)


def kernel(xyz, atom_ids, atom_masses, cond_labels, cond_mask, moments,
           edge_index, t, Wp, bp, We1, be1, We2, be2, Wx, bx, Wh1, bh1, Wh2, bh2):
    L = We1.shape[0]
    src = edge_index[0]
    dst = edge_index[1]

    ids16 = jnp.broadcast_to(atom_ids.astype(i32), (N, 16))
    temb = jnp.broadcast_to(t.reshape(1, 1), (N, 1))
    rest = jnp.concatenate([atom_masses / 12.0, temb, cond_mask, cond_labels,
                            moments / (float(N) * 12.0)], axis=1)
    rest16 = jnp.pad(rest, ((0, 0), (0, 16 - rest.shape[1])))
    wp32 = jnp.pad(Wp, ((0, 32 - Wp.shape[0]), (0, 0)))
    xyz16 = jnp.pad(xyz, ((0, 0), (0, XW - 3)))
    mass16 = jnp.pad(atom_masses, ((0, 0), (0, XW - 1)))
    xyzc = (xyz[:, 0], xyz[:, 1], xyz[:, 2])
    z128 = jnp.zeros((NP, D), f32)

    def wpq(i):
        return (jnp.concatenate([We1[i, :D, :], We1[i, D:2 * D, :]], axis=1),
                jnp.concatenate([jnp.zeros((1, D), f32), be1[i].reshape(1, D)],
                                axis=1))

    w0, b0 = wpq(0)
    h, P, Q = _tc_node0(ids16, rest16, wp32, bp.reshape(1, D), w0, b0)
    x = xyz16
    xc = xyzc
    out16 = None
    for i in range(L):
        last = i == L - 1
        Pg, Qg, d16 = _sc_gather(P, Q, *xc, *xyzc, src, dst)
        res = _tc_edge(last, Pg, Qg, d16,
                       We1[i, 2 * D].reshape(1, D), We2[i],
                       be2[i].reshape(1, D), Wx[i].reshape(1, D),
                       bx[i].reshape(1, 1))
        if not last:
            m, c16 = res
            hagg, xagg = _sc_scatter_full(m, c16, dst, z128)
            h, x, P, Q = _tc_nodei(h, hagg, x, xagg,
                                   Wh1[i, :D], Wh1[i, D:], bh1[i].reshape(1, D),
                                   Wh2[i], bh2[i].reshape(1, D), *wpq(i + 1))
            xc = (x[:, 0], x[:, 1], x[:, 2])
        else:
            (c16,) = res
            xagg = _sc_scatter_x(c16, dst, z128)
            out16 = _tc_final(x, xagg, xyz16, mass16)
    return out16[:, :3]


# speedup vs baseline: 3.3508x; 1.1240x over previous
import functools

import jax
import jax.numpy as jnp
from jax import lax
from jax.experimental import pallas as pl
from jax.experimental.pallas import tpu as pltpu
from jax.experimental.pallas import tpu_sc as plsc

N = 10000
E = 320000
D = 128
XW = 16

NC = 2
NS = 16
NW = NC * NS
EPT = E // NW
CH = 80
NG = CH // 16
NCHUNK = EPT // CH
NP = 10240
NPT = NP // NS

BN = 2000
BE = 2000

f32 = jnp.float32
i32 = jnp.int32


@functools.lru_cache(maxsize=None)
def _mesh():
    return plsc.VectorSubcoreMesh(core_axis_name="c", subcore_axis_name="s",
                                  num_cores=NC, num_subcores=NS)


def _silu(x):
    return x * jax.nn.sigmoid(x)


def _sc_gather_body(p_hbm, q_hbm, xcx_hbm, xcy_hbm, xcz_hbm,
                    xyx_hbm, xyy_hbm, xyz_hbm, src_hbm, dst_hbm,
                    pg_out, qg_out, d_out,
                    idxs0, idxd0, bufp0, bufq0, bufd0,
                    idxs1, idxd1, bufp1, bufq1, bufd1,
                    xc0, xc1, xc2, xy0, xy1, xy2,
                    semp0, semq0, semw0, semp1, semq1, semw1):
    wid = lax.axis_index("s") * NC + lax.axis_index("c")
    base = wid * EPT
    coords = (xc0, xc1, xc2)
    coords0 = (xy0, xy1, xy2)
    for c, src_c in enumerate((xcx_hbm, xcy_hbm, xcz_hbm)):
        pltpu.sync_copy(src_c, coords[c])
    for c, src_c in enumerate((xyx_hbm, xyy_hbm, xyz_hbm)):
        pltpu.sync_copy(src_c, coords0[c])
    lane = lax.iota(i32, 16)
    bufs = ((idxs0, idxd0, bufp0, bufq0, bufd0, semp0, semq0, semw0),
            (idxs1, idxd1, bufp1, bufq1, bufd1, semp1, semq1, semw1))

    def issue(k, B):
        idxs, idxd, bp, bq, bd, sp, sq, sw = B
        off = pl.multiple_of(base + k * CH, 8)
        pltpu.sync_copy(src_hbm.at[pl.ds(off, CH)], idxs)
        pltpu.sync_copy(dst_hbm.at[pl.ds(off, CH)], idxd)
        pltpu.async_copy(p_hbm.at[idxs], bp, sp)
        pltpu.async_copy(q_hbm.at[idxd], bq, sq)

    def waitw(B):
        idxs, idxd, bp, bq, bd, sp, sq, sw = B
        pltpu.make_async_copy(p_hbm.at[pl.ds(0, CH)], bp, sw).wait()
        pltpu.make_async_copy(q_hbm.at[pl.ds(0, CH)], bq, sw).wait()
        pltpu.make_async_copy(d_out.at[pl.ds(0, CH)], bd, sw).wait()

    def finish(k, B):
        idxs, idxd, bp, bq, bd, sp, sq, sw = B
        off = pl.multiple_of(base + k * CH, 8)
        for g in range(NG):
            ivs = idxs[pl.ds(g * 16, 16)]
            ivd = idxd[pl.ds(g * 16, 16)]
            row = lane + g * 16
            for c in range(3):
                cvec = jnp.full((16,), c, i32)
                dx = (plsc.load_gather(coords[c], [ivs])
                      - plsc.load_gather(coords[c], [ivd]))
                plsc.store_scatter(bd, [row, cvec], dx)
                d0 = (plsc.load_gather(coords0[c], [ivs])
                      - plsc.load_gather(coords0[c], [ivd]))
                plsc.store_scatter(bd, [row, cvec + 4], d0)
        pltpu.make_async_copy(p_hbm.at[pl.ds(0, CH)], bp, sp).wait()
        pltpu.make_async_copy(q_hbm.at[pl.ds(0, CH)], bq, sq).wait()
        pltpu.async_copy(bp, pg_out.at[pl.ds(off, CH)], sw)
        pltpu.async_copy(bq, qg_out.at[pl.ds(off, CH)], sw)
        pltpu.async_copy(bd, d_out.at[pl.ds(off, CH)], sw)

    issue(0, bufs[0])

    def body(j, carry):
        for b in (0, 1):
            k = 2 * j + b
            B, NB = bufs[b], bufs[1 - b]
            if b == 0:
                @pl.when(j > 0)
                def _():
                    waitw(NB)
            else:
                waitw(NB)
            issue(k + 1, NB)
            finish(k, B)
        return carry

    lax.fori_loop(0, (NCHUNK - 1) // 2, body, 0)
    waitw(bufs[1])
    finish(NCHUNK - 1, bufs[0])
    waitw(bufs[0])


@functools.lru_cache(maxsize=None)
def _get_sc_gather():
    buf_set = [
        pltpu.VMEM((CH,), i32),
        pltpu.VMEM((CH,), i32),
        pltpu.VMEM((CH, D), f32),
        pltpu.VMEM((CH, D), f32),
        pltpu.VMEM((CH, XW), f32),
    ]
    return pl.kernel(
        _sc_gather_body,
        out_type=[
            jax.ShapeDtypeStruct((E, D), f32),
            jax.ShapeDtypeStruct((E, D), f32),
            jax.ShapeDtypeStruct((E, XW), f32),
        ],
        mesh=_mesh(),
        scratch_types=buf_set + buf_set + [
            pltpu.VMEM((N,), f32),
            pltpu.VMEM((N,), f32),
            pltpu.VMEM((N,), f32),
            pltpu.VMEM((N,), f32),
            pltpu.VMEM((N,), f32),
            pltpu.VMEM((N,), f32),
            pltpu.SemaphoreType.DMA,
            pltpu.SemaphoreType.DMA,
            pltpu.SemaphoreType.DMA,
            pltpu.SemaphoreType.DMA,
            pltpu.SemaphoreType.DMA,
            pltpu.SemaphoreType.DMA,
        ],
        compiler_params=pltpu.CompilerParams(needs_layout_passes=False),
    )


@functools.lru_cache(maxsize=None)
def _make_sc_scatter(with_m: bool):
    out_type = []
    if with_m:
        out_type.append(jax.ShapeDtypeStruct((NC, NP, D), f32))
    out_type.append(jax.ShapeDtypeStruct((NC, NP, D), f32))
    scratch = [
        pltpu.VMEM((CH,), i32),
        pltpu.VMEM((CH, D), f32),
        pltpu.VMEM((CH, XW), f32),
        pltpu.VMEM_SHARED((NP, D), f32),
    ]

    def body(*refs):
        if with_m:
            (m_hbm, c_hbm, dst_hbm, z128_hbm,
             hagg_out, xagg_out, idx_v, bufm, bufc, agg_s) = refs
        else:
            (c_hbm, dst_hbm, z128_hbm,
             xagg_out, idx_v, bufm, bufc, agg_s) = refs
        cid = lax.axis_index("c")
        sid = lax.axis_index("s")
        rows = pl.ds(sid * NPT, NPT)
        base = cid * (E // NC) + sid * EPT

        if with_m:
            pltpu.sync_copy(z128_hbm.at[rows], agg_s.at[rows])
            plsc.subcore_barrier()

            def chunk_m(k, carry):
                off = pl.multiple_of(base + k * CH, 8)
                pltpu.sync_copy(dst_hbm.at[pl.ds(off, CH)], idx_v)
                pltpu.sync_copy(m_hbm.at[pl.ds(off, CH)], bufm)
                pltpu.sync_copy(bufm, agg_s.at[idx_v], add=True)
                return carry

            lax.fori_loop(0, NCHUNK, chunk_m, 0)
            plsc.subcore_barrier()
            pltpu.sync_copy(agg_s.at[rows], hagg_out.at[cid, rows])

        pltpu.sync_copy(z128_hbm.at[rows], agg_s.at[rows])
        pltpu.sync_copy(z128_hbm.at[pl.ds(0, CH)], bufm)
        plsc.subcore_barrier()

        def chunk_x(k, carry):
            off = pl.multiple_of(base + k * CH, 8)
            pltpu.sync_copy(dst_hbm.at[pl.ds(off, CH)], idx_v)
            pltpu.sync_copy(c_hbm.at[pl.ds(off, CH)], bufc)
            for e in range(CH):
                bufm[e, pl.ds(0, XW)] = bufc[e, :]
            pltpu.sync_copy(bufm, agg_s.at[idx_v], add=True)
            return carry

        lax.fori_loop(0, NCHUNK, chunk_x, 0)
        plsc.subcore_barrier()
        pltpu.sync_copy(agg_s.at[rows], xagg_out.at[cid, rows])

    return pl.kernel(body, out_type=out_type, mesh=_mesh(),
                     scratch_types=scratch)


def _sc_gather(*args):
    return _get_sc_gather()(*args)


def _sc_scatter_full(*args):
    return _make_sc_scatter(True)(*args)


def _sc_scatter_x(*args):
    res = _make_sc_scatter(False)(*args)
    return res[0] if isinstance(res, (list, tuple)) else res


def _node0_body(ids_ref, rest_ref, wp_ref, bp_ref, wpq_ref, pqb_ref,
                h_ref, p_ref, q_ref):
    ids = ids_ref[...]
    lane = lax.broadcasted_iota(i32, (BN, 16), 1)
    oh = (ids == lane).astype(f32)
    feat = jnp.concatenate([oh, rest_ref[...]], axis=1)
    h = jnp.dot(feat, wp_ref[...], preferred_element_type=f32) + bp_ref[...]
    pq = jnp.dot(h, wpq_ref[...], preferred_element_type=f32) + pqb_ref[...]
    h_ref[...] = h
    p_ref[...] = pq[:, :D]
    q_ref[...] = pq[:, D:]


def _nodei_body(h_ref, hagg_ref, x_ref, xagg_ref,
                wh1a_ref, wh1b_ref, bh1_ref, wh2_ref, bh2_ref, wpq_ref, pqb_ref,
                h_out, x_out, p_ref, q_ref):
    h = h_ref[...]
    agg = hagg_ref[0] + hagg_ref[1]
    u = _silu(jnp.dot(h, wh1a_ref[...], preferred_element_type=f32)
              + jnp.dot(agg, wh1b_ref[...], preferred_element_type=f32)
              + bh1_ref[...])
    hn = h + jnp.dot(u, wh2_ref[...], preferred_element_type=f32) + bh2_ref[...]
    xn = x_ref[...] + (xagg_ref[0] + xagg_ref[1])[:, :XW]
    pq = jnp.dot(hn, wpq_ref[...], preferred_element_type=f32) + pqb_ref[...]
    h_out[...] = hn
    x_out[...] = xn
    p_ref[...] = pq[:, :D]
    q_ref[...] = pq[:, D:]


def _edge_body(last, pg_ref, qg_ref, d_ref, w257_ref, we2_ref, be2_ref,
               wx_ref, bx_ref, *outs):
    draw = d_ref[...]
    lane = lax.broadcasted_iota(i32, (BE, XW), 1)
    d = jnp.where(lane < 3, draw, 0.0)
    d0 = jnp.where((lane >= 4) & (lane < 7), draw, 0.0)
    dn2 = jnp.sum(d * d, axis=1, keepdims=True)
    a = jnp.sqrt(jnp.sum(d0 * d0, axis=1, keepdims=True))
    pre = pg_ref[...] + qg_ref[...] + a * w257_ref[...]
    u = _silu(pre)
    m = _silu(jnp.dot(u, we2_ref[...], preferred_element_type=f32) + be2_ref[...])
    coef = jnp.sum(m * wx_ref[...], axis=1, keepdims=True) + bx_ref[0, 0]
    dn = jnp.sqrt(dn2) + 1.0
    c16 = d * (coef / (dn * 32.0))
    oi = 0
    if not last:
        outs[oi][...] = m
        oi += 1
    outs[oi][...] = c16


def _final_body(x_ref, xagg_ref, xyz_ref, mass_ref, out_ref):
    x4 = x_ref[...] + (xagg_ref[0] + xagg_ref[1])[:, :XW]
    vel = x4 - xyz_ref[...]
    w = mass_ref[:, 0:1]
    com = jnp.sum(vel * w, axis=0, keepdims=True) / jnp.sum(w)
    out_ref[...] = vel - com


def _full(shape):
    return pl.BlockSpec(shape, lambda i: tuple(0 for _ in shape))


def _rows(shape):
    return pl.BlockSpec(shape, lambda i: (i,) + tuple(0 for _ in shape[1:]))


def _tc_node0(ids16, rest16, wp32, bp, wpq, pqb):
    return pl.pallas_call(
        _node0_body,
        grid=(N // BN,),
        in_specs=[_rows((BN, 16)), _rows((BN, 16)), _full((32, D)),
                  _full((1, D)), _full((D, 2 * D)), _full((1, 2 * D))],
        out_specs=[_rows((BN, D)), _rows((BN, D)), _rows((BN, D))],
        out_shape=[jax.ShapeDtypeStruct((N, D), f32)] * 3,
    )(ids16, rest16, wp32, bp, wpq, pqb)


def _tc_nodei(h, hagg, x, xagg, wh1a, wh1b, bh1, wh2, bh2, wpq, pqb):
    return pl.pallas_call(
        _nodei_body,
        grid=(N // BN,),
        in_specs=[_rows((BN, D)),
                  pl.BlockSpec((NC, BN, D), lambda i: (0, i, 0)),
                  _rows((BN, XW)),
                  pl.BlockSpec((NC, BN, D), lambda i: (0, i, 0)),
                  _full((D, D)), _full((D, D)), _full((1, D)),
                  _full((D, D)), _full((1, D)),
                  _full((D, 2 * D)), _full((1, 2 * D))],
        out_specs=[_rows((BN, D)), _rows((BN, XW)), _rows((BN, D)),
                   _rows((BN, D))],
        out_shape=[jax.ShapeDtypeStruct((N, D), f32),
                   jax.ShapeDtypeStruct((N, XW), f32),
                   jax.ShapeDtypeStruct((N, D), f32),
                   jax.ShapeDtypeStruct((N, D), f32)],
    )(h, hagg, x, xagg, wh1a, wh1b, bh1, wh2, bh2, wpq, pqb)


def _tc_edge(last, pg, qg, d16, w257, we2, be2, wx, bx):
    in_specs = [_rows((BE, D)), _rows((BE, D)), _rows((BE, XW)),
                _full((1, D)), _full((D, D)), _full((1, D)), _full((1, D)),
                _full((1, 1))]
    out_specs, out_shape = [], []
    if not last:
        out_specs.append(_rows((BE, D)))
        out_shape.append(jax.ShapeDtypeStruct((E, D), f32))
    out_specs.append(_rows((BE, XW)))
    out_shape.append(jax.ShapeDtypeStruct((E, XW), f32))
    return pl.pallas_call(
        functools.partial(_edge_body, last),
        grid=(E // BE,),
        in_specs=in_specs,
        out_specs=out_specs,
        out_shape=out_shape,
    )(pg, qg, d16, w257, we2, be2, wx, bx)


def _tc_final(x3, xagg, xyz16, mass16):
    return pl.pallas_call(
        _final_body,
        grid=(1,),
        in_specs=[_rows((N, XW)),
                  pl.BlockSpec((NC, N, D), lambda i: (0, 0, 0)),
                  _rows((N, XW)), _rows((N, XW))],
        out_specs=_rows((N, XW)),
        out_shape=jax.ShapeDtypeStruct((N, XW), f32),
    )(x3, xagg, xyz16, mass16)


def kernel(xyz, atom_ids, atom_masses, cond_labels, cond_mask, moments,
           edge_index, t, Wp, bp, We1, be1, We2, be2, Wx, bx, Wh1, bh1, Wh2, bh2):
    L = We1.shape[0]
    src = edge_index[0]
    dst = edge_index[1]

    ids16 = jnp.broadcast_to(atom_ids.astype(i32), (N, 16))
    temb = jnp.broadcast_to(t.reshape(1, 1), (N, 1))
    rest = jnp.concatenate([atom_masses / 12.0, temb, cond_mask, cond_labels,
                            moments / (float(N) * 12.0)], axis=1)
    rest16 = jnp.pad(rest, ((0, 0), (0, 16 - rest.shape[1])))
    wp32 = jnp.pad(Wp, ((0, 32 - Wp.shape[0]), (0, 0)))
    xyz16 = jnp.pad(xyz, ((0, 0), (0, XW - 3)))
    mass16 = jnp.pad(atom_masses, ((0, 0), (0, XW - 1)))
    xyzc = (xyz[:, 0], xyz[:, 1], xyz[:, 2])
    z128 = jnp.zeros((NP, D), f32)

    def wpq(i):
        return (jnp.concatenate([We1[i, :D, :], We1[i, D:2 * D, :]], axis=1),
                jnp.concatenate([jnp.zeros((1, D), f32), be1[i].reshape(1, D)],
                                axis=1))

    w0, b0 = wpq(0)
    h, P, Q = _tc_node0(ids16, rest16, wp32, bp.reshape(1, D), w0, b0)
    x = xyz16
    xc = xyzc
    out16 = None
    for i in range(L):
        last = i == L - 1
        Pg, Qg, d16 = _sc_gather(P, Q, *xc, *xyzc, src, dst)
        res = _tc_edge(last, Pg, Qg, d16,
                       We1[i, 2 * D].reshape(1, D), We2[i],
                       be2[i].reshape(1, D), Wx[i].reshape(1, D),
                       bx[i].reshape(1, 1))
        if not last:
            m, c16 = res
            hagg, xagg = _sc_scatter_full(m, c16, dst, z128)
            h, x, P, Q = _tc_nodei(h, hagg, x, xagg,
                                   Wh1[i, :D], Wh1[i, D:], bh1[i].reshape(1, D),
                                   Wh2[i], bh2[i].reshape(1, D), *wpq(i + 1))
            xc = (x[:, 0], x[:, 1], x[:, 2])
        else:
            (c16,) = res
            xagg = _sc_scatter_x(c16, dst, z128)
            out16 = _tc_final(x, xagg, xyz16, mass16)
    return out16[:, :3]


# speedup vs baseline: 4.2333x; 1.2634x over previous
import functools

import jax
import jax.numpy as jnp
from jax import lax
from jax.experimental import pallas as pl
from jax.experimental.pallas import tpu as pltpu
from jax.experimental.pallas import tpu_sc as plsc

N = 10000
E = 320000
D = 128
XW = 16

NC = 2
NS = 16
NW = NC * NS
EPT = E // NW
CH = 80
NG = CH // 16
NCHUNK = EPT // CH
NP = 10240
NPT = NP // NS

BN = 2000
BE = 2000

f32 = jnp.float32
i32 = jnp.int32


@functools.lru_cache(maxsize=None)
def _mesh():
    return plsc.VectorSubcoreMesh(core_axis_name="c", subcore_axis_name="s",
                                  num_cores=NC, num_subcores=NS)


def _silu(x):
    return x * jax.nn.sigmoid(x)


def _sc_gather_body(p_hbm, q_hbm, xcx_hbm, xcy_hbm, xcz_hbm,
                    xyx_hbm, xyy_hbm, xyz_hbm, src_hbm, dst_hbm,
                    pg_out, qg_out, d_out,
                    idxs0, idxd0, bufp0, bufq0, bufd0,
                    idxs1, idxd1, bufp1, bufq1, bufd1,
                    xc0, xc1, xc2, xy0, xy1, xy2,
                    semp0, semq0, semw0, semp1, semq1, semw1):
    wid = lax.axis_index("s") * NC + lax.axis_index("c")
    base = wid * EPT
    coords = (xc0, xc1, xc2)
    coords0 = (xy0, xy1, xy2)
    for c, src_c in enumerate((xcx_hbm, xcy_hbm, xcz_hbm)):
        pltpu.sync_copy(src_c, coords[c])
    for c, src_c in enumerate((xyx_hbm, xyy_hbm, xyz_hbm)):
        pltpu.sync_copy(src_c, coords0[c])
    lane = lax.iota(i32, 16)
    bufs = ((idxs0, idxd0, bufp0, bufq0, bufd0, semp0, semq0, semw0),
            (idxs1, idxd1, bufp1, bufq1, bufd1, semp1, semq1, semw1))

    def issue(k, B):
        idxs, idxd, bp, bq, bd, sp, sq, sw = B
        off = pl.multiple_of(base + k * CH, 8)
        pltpu.sync_copy(src_hbm.at[pl.ds(off, CH)], idxs)
        pltpu.sync_copy(dst_hbm.at[pl.ds(off, CH)], idxd)
        pltpu.async_copy(p_hbm.at[idxs], bp, sp)
        pltpu.async_copy(q_hbm.at[idxd], bq, sq)

    def waitw(B):
        idxs, idxd, bp, bq, bd, sp, sq, sw = B
        pltpu.make_async_copy(p_hbm.at[pl.ds(0, CH)], bp, sw).wait()
        pltpu.make_async_copy(q_hbm.at[pl.ds(0, CH)], bq, sw).wait()
        pltpu.make_async_copy(d_out.at[pl.ds(0, CH)], bd, sw).wait()

    def finish(k, B):
        idxs, idxd, bp, bq, bd, sp, sq, sw = B
        off = pl.multiple_of(base + k * CH, 8)
        for g in range(NG):
            ivs = idxs[pl.ds(g * 16, 16)]
            ivd = idxd[pl.ds(g * 16, 16)]
            row = lane + g * 16
            for c in range(3):
                cvec = jnp.full((16,), c, i32)
                dx = (plsc.load_gather(coords[c], [ivs])
                      - plsc.load_gather(coords[c], [ivd]))
                plsc.store_scatter(bd, [row, cvec], dx)
                d0 = (plsc.load_gather(coords0[c], [ivs])
                      - plsc.load_gather(coords0[c], [ivd]))
                plsc.store_scatter(bd, [row, cvec + 4], d0)
        pltpu.make_async_copy(p_hbm.at[pl.ds(0, CH)], bp, sp).wait()
        pltpu.make_async_copy(q_hbm.at[pl.ds(0, CH)], bq, sq).wait()
        pltpu.async_copy(bp, pg_out.at[pl.ds(off, CH)], sw)
        pltpu.async_copy(bq, qg_out.at[pl.ds(off, CH)], sw)
        pltpu.async_copy(bd, d_out.at[pl.ds(off, CH)], sw)

    issue(0, bufs[0])

    def body(j, carry):
        for b in (0, 1):
            k = 2 * j + b
            B, NB = bufs[b], bufs[1 - b]
            if b == 0:
                @pl.when(j > 0)
                def _():
                    waitw(NB)
            else:
                waitw(NB)
            issue(k + 1, NB)
            finish(k, B)
        return carry

    lax.fori_loop(0, (NCHUNK - 1) // 2, body, 0)
    waitw(bufs[1])
    finish(NCHUNK - 1, bufs[0])
    waitw(bufs[0])


@functools.lru_cache(maxsize=None)
def _get_sc_gather():
    buf_set = [
        pltpu.VMEM((CH,), i32),
        pltpu.VMEM((CH,), i32),
        pltpu.VMEM((CH, D), f32),
        pltpu.VMEM((CH, D), f32),
        pltpu.VMEM((CH, XW), f32),
    ]
    return pl.kernel(
        _sc_gather_body,
        out_type=[
            jax.ShapeDtypeStruct((E, D), f32),
            jax.ShapeDtypeStruct((E, D), f32),
            jax.ShapeDtypeStruct((E, XW), f32),
        ],
        mesh=_mesh(),
        scratch_types=buf_set + buf_set + [
            pltpu.VMEM((N,), f32),
            pltpu.VMEM((N,), f32),
            pltpu.VMEM((N,), f32),
            pltpu.VMEM((N,), f32),
            pltpu.VMEM((N,), f32),
            pltpu.VMEM((N,), f32),
            pltpu.SemaphoreType.DMA,
            pltpu.SemaphoreType.DMA,
            pltpu.SemaphoreType.DMA,
            pltpu.SemaphoreType.DMA,
            pltpu.SemaphoreType.DMA,
            pltpu.SemaphoreType.DMA,
        ],
        compiler_params=pltpu.CompilerParams(needs_layout_passes=False),
    )


@functools.lru_cache(maxsize=None)
def _make_sc_scatter(with_m: bool):
    out_type = []
    if with_m:
        out_type.append(jax.ShapeDtypeStruct((NC, NP, D), f32))
    out_type.append(jax.ShapeDtypeStruct((NC, NP, D), f32))
    buf_set = [
        pltpu.VMEM((CH,), i32),
        pltpu.VMEM((CH, D), f32),
        pltpu.VMEM((CH, XW), f32),
    ]
    scratch = buf_set + buf_set + [
        pltpu.VMEM_SHARED((NP, D), f32),
        pltpu.SemaphoreType.DMA,
        pltpu.SemaphoreType.DMA,
        pltpu.SemaphoreType.DMA,
        pltpu.SemaphoreType.DMA,
    ]

    def body(*refs):
        if with_m:
            (m_hbm, c_hbm, dst_hbm, z128_hbm, hagg_out, xagg_out,
             idx0, bufm0, bufc0, idx1, bufm1, bufc1, agg_s,
             sl0, sw0, sl1, sw1) = refs
        else:
            (c_hbm, dst_hbm, z128_hbm, xagg_out,
             idx0, bufm0, bufc0, idx1, bufm1, bufc1, agg_s,
             sl0, sw0, sl1, sw1) = refs
        cid = lax.axis_index("c")
        sid = lax.axis_index("s")
        rows = pl.ds(sid * NPT, NPT)
        base = cid * (E // NC) + sid * EPT
        bufs = ((idx0, bufm0, bufc0, sl0, sw0),
                (idx1, bufm1, bufc1, sl1, sw1))

        def run_phase(data_hbm, is_m):
            def issue(k, B):
                idx, bm, bc, sl, sw = B
                off = pl.multiple_of(base + k * CH, 8)
                pltpu.async_copy(dst_hbm.at[pl.ds(off, CH)], idx, sl)
                pltpu.async_copy(data_hbm.at[pl.ds(off, CH)],
                                 bm if is_m else bc, sl)

            def finish(k, B):
                idx, bm, bc, sl, sw = B
                pltpu.make_async_copy(dst_hbm.at[pl.ds(0, CH)], idx, sl).wait()
                if is_m:
                    pltpu.make_async_copy(m_hbm.at[pl.ds(0, CH)], bm, sl).wait()
                else:
                    pltpu.make_async_copy(data_hbm.at[pl.ds(0, CH)], bc,
                                          sl).wait()
                    for e in range(CH):
                        bm[e, pl.ds(0, XW)] = bc[e, :]
                pltpu.async_copy(bm, agg_s.at[idx], sw, add=True)

            def waitw(B):
                idx, bm, bc, sl, sw = B
                pltpu.make_async_copy(z128_hbm.at[pl.ds(0, CH)], bm, sw).wait()

            issue(0, bufs[0])

            def loop(j, carry):
                for b in (0, 1):
                    k = 2 * j + b
                    B, NB = bufs[b], bufs[1 - b]
                    if b == 0:
                        @pl.when(j > 0)
                        def _():
                            waitw(NB)
                    else:
                        waitw(NB)
                    issue(k + 1, NB)
                    finish(k, B)
                return carry

            lax.fori_loop(0, (NCHUNK - 1) // 2, loop, 0)
            waitw(bufs[1])
            finish(NCHUNK - 1, bufs[0])
            waitw(bufs[0])

        if with_m:
            pltpu.sync_copy(z128_hbm.at[rows], agg_s.at[rows])
            plsc.subcore_barrier()
            run_phase(m_hbm, True)
            plsc.subcore_barrier()
            pltpu.sync_copy(agg_s.at[rows], hagg_out.at[cid, rows])

        pltpu.sync_copy(z128_hbm.at[rows], agg_s.at[rows])
        pltpu.sync_copy(z128_hbm.at[pl.ds(0, CH)], bufm0)
        pltpu.sync_copy(z128_hbm.at[pl.ds(0, CH)], bufm1)
        plsc.subcore_barrier()
        run_phase(c_hbm, False)
        plsc.subcore_barrier()
        pltpu.sync_copy(agg_s.at[rows], xagg_out.at[cid, rows])

    return pl.kernel(body, out_type=out_type, mesh=_mesh(),
                     scratch_types=scratch)


def _sc_gather(*args):
    return _get_sc_gather()(*args)


def _sc_scatter_full(*args):
    return _make_sc_scatter(True)(*args)


def _sc_scatter_x(*args):
    res = _make_sc_scatter(False)(*args)
    return res[0] if isinstance(res, (list, tuple)) else res


def _node0_body(ids_ref, rest_ref, wp_ref, bp_ref, wpq_ref, pqb_ref,
                h_ref, p_ref, q_ref):
    ids = ids_ref[...]
    lane = lax.broadcasted_iota(i32, (BN, 16), 1)
    oh = (ids == lane).astype(f32)
    feat = jnp.concatenate([oh, rest_ref[...]], axis=1)
    h = jnp.dot(feat, wp_ref[...], preferred_element_type=f32) + bp_ref[...]
    pq = jnp.dot(h, wpq_ref[...], preferred_element_type=f32) + pqb_ref[...]
    h_ref[...] = h
    p_ref[...] = pq[:, :D]
    q_ref[...] = pq[:, D:]


def _nodei_body(h_ref, hagg_ref, x_ref, xagg_ref,
                wh1a_ref, wh1b_ref, bh1_ref, wh2_ref, bh2_ref, wpq_ref, pqb_ref,
                h_out, x_out, p_ref, q_ref):
    h = h_ref[...]
    agg = hagg_ref[0] + hagg_ref[1]
    u = _silu(jnp.dot(h, wh1a_ref[...], preferred_element_type=f32)
              + jnp.dot(agg, wh1b_ref[...], preferred_element_type=f32)
              + bh1_ref[...])
    hn = h + jnp.dot(u, wh2_ref[...], preferred_element_type=f32) + bh2_ref[...]
    xn = x_ref[...] + (xagg_ref[0] + xagg_ref[1])[:, :XW]
    pq = jnp.dot(hn, wpq_ref[...], preferred_element_type=f32) + pqb_ref[...]
    h_out[...] = hn
    x_out[...] = xn
    p_ref[...] = pq[:, :D]
    q_ref[...] = pq[:, D:]


def _edge_body(last, pg_ref, qg_ref, d_ref, w257_ref, we2_ref, be2_ref,
               wx_ref, bx_ref, *outs):
    draw = d_ref[...]
    lane = lax.broadcasted_iota(i32, (BE, XW), 1)
    d = jnp.where(lane < 3, draw, 0.0)
    d0 = jnp.where((lane >= 4) & (lane < 7), draw, 0.0)
    dn2 = jnp.sum(d * d, axis=1, keepdims=True)
    a = jnp.sqrt(jnp.sum(d0 * d0, axis=1, keepdims=True))
    pre = pg_ref[...] + qg_ref[...] + a * w257_ref[...]
    u = _silu(pre)
    m = _silu(jnp.dot(u, we2_ref[...], preferred_element_type=f32) + be2_ref[...])
    coef = jnp.sum(m * wx_ref[...], axis=1, keepdims=True) + bx_ref[0, 0]
    dn = jnp.sqrt(dn2) + 1.0
    c16 = d * (coef / (dn * 32.0))
    oi = 0
    if not last:
        outs[oi][...] = m
        oi += 1
    outs[oi][...] = c16


def _final_body(x_ref, xagg_ref, xyz_ref, mass_ref, out_ref):
    x4 = x_ref[...] + (xagg_ref[0] + xagg_ref[1])[:, :XW]
    vel = x4 - xyz_ref[...]
    w = mass_ref[:, 0:1]
    com = jnp.sum(vel * w, axis=0, keepdims=True) / jnp.sum(w)
    out_ref[...] = vel - com


def _full(shape):
    return pl.BlockSpec(shape, lambda i: tuple(0 for _ in shape))


def _rows(shape):
    return pl.BlockSpec(shape, lambda i: (i,) + tuple(0 for _ in shape[1:]))


def _tc_node0(ids16, rest16, wp32, bp, wpq, pqb):
    return pl.pallas_call(
        _node0_body,
        grid=(N // BN,),
        in_specs=[_rows((BN, 16)), _rows((BN, 16)), _full((32, D)),
                  _full((1, D)), _full((D, 2 * D)), _full((1, 2 * D))],
        out_specs=[_rows((BN, D)), _rows((BN, D)), _rows((BN, D))],
        out_shape=[jax.ShapeDtypeStruct((N, D), f32)] * 3,
    )(ids16, rest16, wp32, bp, wpq, pqb)


def _tc_nodei(h, hagg, x, xagg, wh1a, wh1b, bh1, wh2, bh2, wpq, pqb):
    return pl.pallas_call(
        _nodei_body,
        grid=(N // BN,),
        in_specs=[_rows((BN, D)),
                  pl.BlockSpec((NC, BN, D), lambda i: (0, i, 0)),
                  _rows((BN, XW)),
                  pl.BlockSpec((NC, BN, D), lambda i: (0, i, 0)),
                  _full((D, D)), _full((D, D)), _full((1, D)),
                  _full((D, D)), _full((1, D)),
                  _full((D, 2 * D)), _full((1, 2 * D))],
        out_specs=[_rows((BN, D)), _rows((BN, XW)), _rows((BN, D)),
                   _rows((BN, D))],
        out_shape=[jax.ShapeDtypeStruct((N, D), f32),
                   jax.ShapeDtypeStruct((N, XW), f32),
                   jax.ShapeDtypeStruct((N, D), f32),
                   jax.ShapeDtypeStruct((N, D), f32)],
    )(h, hagg, x, xagg, wh1a, wh1b, bh1, wh2, bh2, wpq, pqb)


def _tc_edge(last, pg, qg, d16, w257, we2, be2, wx, bx):
    in_specs = [_rows((BE, D)), _rows((BE, D)), _rows((BE, XW)),
                _full((1, D)), _full((D, D)), _full((1, D)), _full((1, D)),
                _full((1, 1))]
    out_specs, out_shape = [], []
    if not last:
        out_specs.append(_rows((BE, D)))
        out_shape.append(jax.ShapeDtypeStruct((E, D), f32))
    out_specs.append(_rows((BE, XW)))
    out_shape.append(jax.ShapeDtypeStruct((E, XW), f32))
    return pl.pallas_call(
        functools.partial(_edge_body, last),
        grid=(E // BE,),
        in_specs=in_specs,
        out_specs=out_specs,
        out_shape=out_shape,
    )(pg, qg, d16, w257, we2, be2, wx, bx)


def _tc_final(x3, xagg, xyz16, mass16):
    return pl.pallas_call(
        _final_body,
        grid=(1,),
        in_specs=[_rows((N, XW)),
                  pl.BlockSpec((NC, N, D), lambda i: (0, 0, 0)),
                  _rows((N, XW)), _rows((N, XW))],
        out_specs=_rows((N, XW)),
        out_shape=jax.ShapeDtypeStruct((N, XW), f32),
    )(x3, xagg, xyz16, mass16)


def kernel(xyz, atom_ids, atom_masses, cond_labels, cond_mask, moments,
           edge_index, t, Wp, bp, We1, be1, We2, be2, Wx, bx, Wh1, bh1, Wh2, bh2):
    L = We1.shape[0]
    src = edge_index[0]
    dst = edge_index[1]

    ids16 = jnp.broadcast_to(atom_ids.astype(i32), (N, 16))
    temb = jnp.broadcast_to(t.reshape(1, 1), (N, 1))
    rest = jnp.concatenate([atom_masses / 12.0, temb, cond_mask, cond_labels,
                            moments / (float(N) * 12.0)], axis=1)
    rest16 = jnp.pad(rest, ((0, 0), (0, 16 - rest.shape[1])))
    wp32 = jnp.pad(Wp, ((0, 32 - Wp.shape[0]), (0, 0)))
    xyz16 = jnp.pad(xyz, ((0, 0), (0, XW - 3)))
    mass16 = jnp.pad(atom_masses, ((0, 0), (0, XW - 1)))
    xyzc = (xyz[:, 0], xyz[:, 1], xyz[:, 2])
    z128 = jnp.zeros((NP, D), f32)

    def wpq(i):
        return (jnp.concatenate([We1[i, :D, :], We1[i, D:2 * D, :]], axis=1),
                jnp.concatenate([jnp.zeros((1, D), f32), be1[i].reshape(1, D)],
                                axis=1))

    w0, b0 = wpq(0)
    h, P, Q = _tc_node0(ids16, rest16, wp32, bp.reshape(1, D), w0, b0)
    x = xyz16
    xc = xyzc
    out16 = None
    for i in range(L):
        last = i == L - 1
        Pg, Qg, d16 = _sc_gather(P, Q, *xc, *xyzc, src, dst)
        res = _tc_edge(last, Pg, Qg, d16,
                       We1[i, 2 * D].reshape(1, D), We2[i],
                       be2[i].reshape(1, D), Wx[i].reshape(1, D),
                       bx[i].reshape(1, 1))
        if not last:
            m, c16 = res
            hagg, xagg = _sc_scatter_full(m, c16, dst, z128)
            h, x, P, Q = _tc_nodei(h, hagg, x, xagg,
                                   Wh1[i, :D], Wh1[i, D:], bh1[i].reshape(1, D),
                                   Wh2[i], bh2[i].reshape(1, D), *wpq(i + 1))
            xc = (x[:, 0], x[:, 1], x[:, 2])
        else:
            (c16,) = res
            xagg = _sc_scatter_x(c16, dst, z128)
            out16 = _tc_final(x, xagg, xyz16, mass16)
    return out16[:, :3]
